# Initial kernel scaffold; baseline (speedup 1.0000x reference)
import functools

import jax
import jax.numpy as jnp
from jax import lax
from jax.experimental import pallas as pl
from jax.experimental.pallas import tpu as pltpu
from jax.experimental.pallas import tpu_sc as plsc

K9 = 9
N = 4096
B = 4
F = 32
QB = 256
QC = 512


def _knn_body(xq_ref, yq_ref, xk_ref, yk_ref, idx_ref, xs_ref, ys_ref):
    qx = xq_ref[0, :]
    qy = yq_ref[0, :]
    kx = xk_ref[0, :]
    ky = yk_ref[0, :]

    qsq = qx * qx + qy * qy
    ksq = kx * kx + ky * ky
    dot = qx[:, None] * kx[None, :] + qy[:, None] * ky[None, :]
    d2 = qsq[:, None] + ksq[None, :] - 2.0 * dot

    iota = lax.broadcasted_iota(jnp.int32, (QB, N), 1)
    kxb = jnp.broadcast_to(kx[None, :], (QB, N))
    kyb = jnp.broadcast_to(ky[None, :], (QB, N))
    inf = jnp.float32(jnp.inf)
    cur = d2
    for r in range(K9):
        m = jnp.min(cur, axis=1, keepdims=True)
        hit = cur == m
        j = jnp.min(jnp.where(hit, iota, N), axis=1, keepdims=True)
        sel = iota == j
        xj = jnp.min(jnp.where(sel, kxb, inf), axis=1)
        yj = jnp.min(jnp.where(sel, kyb, inf), axis=1)
        idx_ref[0, :, r] = j[:, 0]
        xs_ref[0, :, r] = xj
        ys_ref[0, :, r] = yj
        cur = jnp.where(sel, inf, cur)


def _knn_tc(x, y):
    grid = (B, N // QB)
    q_spec = pl.BlockSpec((1, QB), lambda b, q: (b, q))
    k_spec = pl.BlockSpec((1, N), lambda b, q: (b, 0))
    o_spec = pl.BlockSpec((1, QB, K9), lambda b, q: (b, q, 0))
    return pl.pallas_call(
        _knn_body,
        grid=grid,
        in_specs=[q_spec, q_spec, k_spec, k_spec],
        out_specs=[o_spec, o_spec, o_spec],
        out_shape=[
            jax.ShapeDtypeStruct((B, N, K9), jnp.int32),
            jax.ShapeDtypeStruct((B, N, K9), jnp.float32),
            jax.ShapeDtypeStruct((B, N, K9), jnp.float32),
        ],
    )(x, y, x, y)


_SC_INFO = plsc.get_sparse_core_info()
_NC = _SC_INFO.num_cores
_NS = _SC_INFO.num_subcores
_NW = _NC * _NS
_ROWS = B * N * K9
_RPW = _ROWS // _NW
_CH = 1152


@functools.partial(
    pl.kernel,
    out_type=jax.ShapeDtypeStruct((_ROWS, F), jnp.float32),
    mesh=plsc.VectorSubcoreMesh(core_axis_name="c", subcore_axis_name="s"),
    scratch_types=[
        pltpu.VMEM((_CH,), jnp.int32),
        pltpu.VMEM((_CH, F), jnp.float32),
        pltpu.SemaphoreType.DMA,
    ],
)
def _gather_sc(tab_hbm, idx_hbm, out_hbm, idx_v, rows_v, sem):
    wid = lax.axis_index("s") * _NC + lax.axis_index("c")
    base = wid * _RPW
    for c in range(_RPW // _CH):
        off = base + c * _CH
        pltpu.sync_copy(idx_hbm.at[pl.ds(off, _CH)], idx_v)
        pltpu.async_copy(tab_hbm.at[idx_v], rows_v, sem).wait()
        pltpu.sync_copy(rows_v, out_hbm.at[pl.ds(off, _CH)])


def _mlp_body(x_ref, y_ref, nf_ref, wx_ref, wy_ref, wv_ref, out_ref, ggf_ref):
    X = x_ref[0]
    Y = y_ref[0]
    NF = nf_ref[0]
    Wx = wx_ref[...]
    Wy = wy_ref[...]
    w0p = wv_ref[0]
    w1p = wv_ref[1]
    wns = wv_ref[2]
    bb = wv_ref[3]

    P = (lax.dot(X, Wx, precision=lax.Precision.HIGHEST,
                 preferred_element_type=jnp.float32)
         + lax.dot(Y, Wy, precision=lax.Precision.HIGHEST,
                   preferred_element_type=jnp.float32))
    Nrm = jnp.sqrt(X * X + Y * Y)
    R3 = (P[:, None, :]
          + X[..., None] * w0p[None, None, :]
          + Y[..., None] * w1p[None, None, :]
          + Nrm[..., None] * wns[None, None, :]
          + bb[None, None, :])
    R3 = jnp.maximum(R3, 0.0)
    out_ref[0] = jnp.concatenate([NF, R3], axis=-1)

    n = jnp.float32(K9)
    sx = jnp.sum(X, axis=1)
    sy = jnp.sum(Y, axis=1)
    sxy = jnp.sum(X * Y, axis=1)
    sxx = jnp.sum(X * X, axis=1)
    m = (n * sxy - sx * sy) / (n * sxx - sx * sx + 1e-8)
    xm = X - (sx / n)[:, None]
    ym = Y - (sy / n)[:, None]
    r = jnp.sum(xm * ym, axis=1) / (
        jnp.sqrt(jnp.sum(xm * xm, axis=1) * jnp.sum(ym * ym, axis=1)) + 1e-8)
    ggf_ref[0, 0, :] = m
    ggf_ref[0, 1, :] = 1.0 - r


def _mlp_tc(xs, ys, nf, Wx, Wy, wv):
    grid = (B, N // QC)
    c_spec = pl.BlockSpec((1, QC, K9), lambda b, q: (b, q, 0))
    nf_spec = pl.BlockSpec((1, QC, K9, F), lambda b, q: (b, q, 0, 0))
    wxy_spec = pl.BlockSpec((K9, F), lambda b, q: (0, 0))
    wv_spec = pl.BlockSpec((4, F), lambda b, q: (0, 0))
    out_spec = pl.BlockSpec((1, QC, K9, 2 * F), lambda b, q: (b, q, 0, 0))
    ggf_spec = pl.BlockSpec((1, 2, QC), lambda b, q: (b, 0, q))
    return pl.pallas_call(
        _mlp_body,
        grid=grid,
        in_specs=[c_spec, c_spec, nf_spec, wxy_spec, wxy_spec, wv_spec],
        out_specs=[out_spec, ggf_spec],
        out_shape=[
            jax.ShapeDtypeStruct((B, N, K9, 2 * F), jnp.float32),
            jax.ShapeDtypeStruct((B, 2, N), jnp.float32),
        ],
    )(xs, ys, nf, Wx, Wy, wv)


def kernel(pc, feats, W, b):
    x = pc[..., 0]
    y = pc[..., 1]

    idx9, xs, ys = _knn_tc(x, y)

    offs = (jnp.arange(B, dtype=jnp.int32) * N)[:, None, None]
    gidx = (idx9 + offs).reshape(-1)
    nf = _gather_sc(feats.reshape(B * N, F), gidx)
    nf = nf.reshape(B, N, K9, F)

    Wx = W[2::3]
    Wy = W[3::3]
    wv = jnp.stack([
        W[0] - jnp.sum(Wx, axis=0),
        W[1] - jnp.sum(Wy, axis=0),
        jnp.sum(W[4::3], axis=0),
        b,
    ])

    out, ggf2 = _mlp_tc(xs, ys, nf, Wx, Wy, wv)
    ggf = jnp.moveaxis(ggf2, 1, 2)[:, :, None, :]
    return out, ggf


# speedup vs baseline: 14.3035x; 14.3035x over previous
import functools

import jax
import jax.numpy as jnp
from jax import lax
from jax.experimental import pallas as pl
from jax.experimental.pallas import tpu as pltpu
from jax.experimental.pallas import tpu_sc as plsc

K9 = 9
N = 4096
B = 4
F = 32
QB = 256
QC = 512


def _knn_body(xq_ref, yq_ref, xk_ref, yk_ref, qp_ref, kp_ref,
              idx_ref, xs_ref, ys_ref):
    qx = xq_ref[0, 0, :]
    qy = yq_ref[0, 0, :]
    kx = xk_ref[0, 0, :]
    ky = yk_ref[0, 0, :]

    qsq = qx * qx + qy * qy
    ksq = kx * kx + ky * ky
    dot = lax.dot_general(qp_ref[0], kp_ref[0],
                          (((1,), (1,)), ((), ())),
                          preferred_element_type=jnp.float32)
    d2 = qsq[:, None] + ksq[None, :] - 2.0 * dot

    iota = lax.broadcasted_iota(jnp.int32, (QB, N), 1)
    kxb = jnp.broadcast_to(kx[None, :], (QB, N))
    kyb = jnp.broadcast_to(ky[None, :], (QB, N))
    inf = jnp.float32(jnp.inf)
    cur = d2
    for r in range(K9):
        m = jnp.min(cur, axis=1, keepdims=True)
        hit = cur == m
        j = jnp.min(jnp.where(hit, iota, N), axis=1, keepdims=True)
        sel = iota == j
        xj = jnp.min(jnp.where(sel, kxb, inf), axis=1)
        yj = jnp.min(jnp.where(sel, kyb, inf), axis=1)
        idx_ref[0, :, r] = j[:, 0]
        xs_ref[0, :, r] = xj
        ys_ref[0, :, r] = yj
        cur = jnp.where(sel, inf, cur)


def _knn_tc(x, y, pc):
    x = x.reshape(B, 1, N)
    y = y.reshape(B, 1, N)
    grid = (B, N // QB)
    q_spec = pl.BlockSpec((1, 1, QB), lambda b, q: (b, 0, q))
    k_spec = pl.BlockSpec((1, 1, N), lambda b, q: (b, 0, 0))
    qp_spec = pl.BlockSpec((1, QB, 2), lambda b, q: (b, q, 0))
    kp_spec = pl.BlockSpec((1, N, 2), lambda b, q: (b, 0, 0))
    o_spec = pl.BlockSpec((1, QB, K9), lambda b, q: (b, q, 0))
    return pl.pallas_call(
        _knn_body,
        grid=grid,
        in_specs=[q_spec, q_spec, k_spec, k_spec, qp_spec, kp_spec],
        out_specs=[o_spec, o_spec, o_spec],
        out_shape=[
            jax.ShapeDtypeStruct((B, N, K9), jnp.int32),
            jax.ShapeDtypeStruct((B, N, K9), jnp.float32),
            jax.ShapeDtypeStruct((B, N, K9), jnp.float32),
        ],
    )(x, y, x, y, pc, pc)


_NC = 2
_NS = 16
_NW = _NC * _NS
_ROWS = B * N * K9
_RPW = _ROWS // _NW
_CH = 1152


@functools.cache
def _make_gather_sc():
    @functools.partial(
        pl.kernel,
        out_type=jax.ShapeDtypeStruct((_ROWS, F), jnp.float32),
        mesh=plsc.VectorSubcoreMesh(core_axis_name="c", subcore_axis_name="s",
                                    num_cores=_NC, num_subcores=_NS),
        scratch_types=[
            pltpu.VMEM((_CH,), jnp.int32),
            pltpu.VMEM((_CH, F), jnp.float32),
            pltpu.SemaphoreType.DMA,
        ],
        compiler_params=pltpu.CompilerParams(use_tc_tiling_on_sc=False),
    )
    def gather(tab_hbm, idx_hbm, out_hbm, idx_v, rows_v, sem):
        wid = lax.axis_index("s") * _NC + lax.axis_index("c")
        base = wid * _RPW
        for c in range(_RPW // _CH):
            off = base + c * _CH
            pltpu.sync_copy(idx_hbm.at[pl.ds(off, _CH)], idx_v)
            pltpu.async_copy(tab_hbm.at[idx_v], rows_v, sem).wait()
            pltpu.sync_copy(rows_v, out_hbm.at[pl.ds(off, _CH)])

    return gather


def _gather_sc(tab, idx):
    return _make_gather_sc()(tab, idx)


def _mlp_body(x_ref, y_ref, nf_ref, wx_ref, wy_ref, wv_ref, out_ref, ggf_ref):
    X = x_ref[0]
    Y = y_ref[0]
    NF = nf_ref[0]
    Wx = wx_ref[...]
    Wy = wy_ref[...]
    w0p = wv_ref[0]
    w1p = wv_ref[1]
    wns = wv_ref[2]
    bb = wv_ref[3]

    P = (lax.dot(X, Wx, precision=lax.Precision.HIGHEST,
                 preferred_element_type=jnp.float32)
         + lax.dot(Y, Wy, precision=lax.Precision.HIGHEST,
                   preferred_element_type=jnp.float32))
    Nrm = jnp.sqrt(X * X + Y * Y)
    R3 = (P[:, None, :]
          + X[..., None] * w0p[None, None, :]
          + Y[..., None] * w1p[None, None, :]
          + Nrm[..., None] * wns[None, None, :]
          + bb[None, None, :])
    R3 = jnp.maximum(R3, 0.0)
    out_ref[0] = jnp.concatenate([NF, R3], axis=-1)

    n = jnp.float32(K9)
    sx = jnp.sum(X, axis=1)
    sy = jnp.sum(Y, axis=1)
    sxy = jnp.sum(X * Y, axis=1)
    sxx = jnp.sum(X * X, axis=1)
    m = (n * sxy - sx * sy) / (n * sxx - sx * sx + 1e-8)
    xm = X - (sx / n)[:, None]
    ym = Y - (sy / n)[:, None]
    r = jnp.sum(xm * ym, axis=1) / (
        jnp.sqrt(jnp.sum(xm * xm, axis=1) * jnp.sum(ym * ym, axis=1)) + 1e-8)
    ggf_ref[0, 0, :] = m
    ggf_ref[0, 1, :] = 1.0 - r


def _mlp_tc(xs, ys, nf, Wx, Wy, wv):
    grid = (B, N // QC)
    c_spec = pl.BlockSpec((1, QC, K9), lambda b, q: (b, q, 0))
    nf_spec = pl.BlockSpec((1, QC, K9, F), lambda b, q: (b, q, 0, 0))
    wxy_spec = pl.BlockSpec((K9, F), lambda b, q: (0, 0))
    wv_spec = pl.BlockSpec((4, F), lambda b, q: (0, 0))
    out_spec = pl.BlockSpec((1, QC, K9, 2 * F), lambda b, q: (b, q, 0, 0))
    ggf_spec = pl.BlockSpec((1, 2, QC), lambda b, q: (b, 0, q))
    return pl.pallas_call(
        _mlp_body,
        grid=grid,
        in_specs=[c_spec, c_spec, nf_spec, wxy_spec, wxy_spec, wv_spec],
        out_specs=[out_spec, ggf_spec],
        out_shape=[
            jax.ShapeDtypeStruct((B, N, K9, 2 * F), jnp.float32),
            jax.ShapeDtypeStruct((B, 2, N), jnp.float32),
        ],
    )(xs, ys, nf, Wx, Wy, wv)


def kernel(pc, feats, W, b):
    x = pc[..., 0]
    y = pc[..., 1]

    idx9, xs, ys = _knn_tc(x, y, pc)

    offs = (jnp.arange(B, dtype=jnp.int32) * N)[:, None, None]
    gidx = (idx9 + offs).reshape(-1)
    nf = _gather_sc(feats.reshape(B * N, F), gidx)
    nf = nf.reshape(B, N, K9, F)

    Wx = W[2::3]
    Wy = W[3::3]
    wv = jnp.stack([
        W[0] - jnp.sum(Wx, axis=0),
        W[1] - jnp.sum(Wy, axis=0),
        jnp.sum(W[4::3], axis=0),
        b,
    ])

    out, ggf2 = _mlp_tc(xs, ys, nf, Wx, Wy, wv)
    ggf = jnp.moveaxis(ggf2, 1, 2)[:, :, None, :]
    return out, ggf


# speedup vs baseline: 14.8980x; 1.0416x over previous
import functools

import jax
import jax.numpy as jnp
from jax import lax
from jax.experimental import pallas as pl
from jax.experimental.pallas import tpu as pltpu
from jax.experimental.pallas import tpu_sc as plsc

K9 = 9
N = 4096
B = 4
F = 32
QB = 256
QC = 512


def _knn_body(xq_ref, yq_ref, xk_ref, yk_ref, qp_ref, kp_ref, idx_ref):
    qx = xq_ref[0, 0, :]
    qy = yq_ref[0, 0, :]
    kx = xk_ref[0, 0, :]
    ky = yk_ref[0, 0, :]

    qsq = qx * qx + qy * qy
    ksq = kx * kx + ky * ky
    dot = lax.dot_general(qp_ref[0], kp_ref[0],
                          (((1,), (1,)), ((), ())),
                          preferred_element_type=jnp.float32)
    d2 = qsq[:, None] + ksq[None, :] - 2.0 * dot

    iota = lax.broadcasted_iota(jnp.int32, (QB, N), 1)
    inf = jnp.float32(jnp.inf)
    cur = d2
    for r in range(K9):
        m = jnp.min(cur, axis=1, keepdims=True)
        hit = cur == m
        j = jnp.min(jnp.where(hit, iota, N), axis=1, keepdims=True)
        idx_ref[0, :, r] = j[:, 0]
        cur = jnp.where(iota == j, inf, cur)


def _knn_tc(x, y, pc):
    x = x.reshape(B, 1, N)
    y = y.reshape(B, 1, N)
    grid = (B, N // QB)
    q_spec = pl.BlockSpec((1, 1, QB), lambda b, q: (b, 0, q))
    k_spec = pl.BlockSpec((1, 1, N), lambda b, q: (b, 0, 0))
    qp_spec = pl.BlockSpec((1, QB, 2), lambda b, q: (b, q, 0))
    kp_spec = pl.BlockSpec((1, N, 2), lambda b, q: (b, 0, 0))
    o_spec = pl.BlockSpec((1, QB, K9), lambda b, q: (b, q, 0))
    return pl.pallas_call(
        _knn_body,
        grid=grid,
        in_specs=[q_spec, q_spec, k_spec, k_spec, qp_spec, kp_spec],
        out_specs=o_spec,
        out_shape=jax.ShapeDtypeStruct((B, N, K9), jnp.int32),
    )(x, y, x, y, pc, pc)


_NC = 2
_NS = 16
_NW = _NC * _NS
_ROWS = B * N * K9
_RPW = _ROWS // _NW
_CH = 1152
_TW = 48


@functools.cache
def _make_gather_sc():
    @functools.partial(
        pl.kernel,
        out_type=jax.ShapeDtypeStruct((_ROWS, _TW), jnp.float32),
        mesh=plsc.VectorSubcoreMesh(core_axis_name="c", subcore_axis_name="s",
                                    num_cores=_NC, num_subcores=_NS),
        scratch_types=[
            pltpu.VMEM((_CH,), jnp.int32),
            pltpu.VMEM((_CH, _TW), jnp.float32),
            pltpu.SemaphoreType.DMA,
        ],
        compiler_params=pltpu.CompilerParams(use_tc_tiling_on_sc=False),
    )
    def gather(tab_hbm, idx_hbm, out_hbm, idx_v, rows_v, sem):
        wid = lax.axis_index("s") * _NC + lax.axis_index("c")
        base = wid * _RPW
        for c in range(_RPW // _CH):
            off = base + c * _CH
            pltpu.sync_copy(idx_hbm.at[pl.ds(off, _CH)], idx_v)
            pltpu.async_copy(tab_hbm.at[idx_v], rows_v, sem).wait()
            pltpu.sync_copy(rows_v, out_hbm.at[pl.ds(off, _CH)])

    return gather


def _gather_sc(tab, idx):
    return _make_gather_sc()(tab, idx)


def _mlp_body(g_ref, wx_ref, wy_ref, wv_ref, out_ref, ggf_ref):
    G = g_ref[0]
    NF = G[:, :, :F]
    X = G[:, :, F]
    Y = G[:, :, F + 1]
    Wx = wx_ref[...]
    Wy = wy_ref[...]
    w0p = wv_ref[0]
    w1p = wv_ref[1]
    wns = wv_ref[2]
    bb = wv_ref[3]

    P = (lax.dot(X, Wx, precision=lax.Precision.HIGHEST,
                 preferred_element_type=jnp.float32)
         + lax.dot(Y, Wy, precision=lax.Precision.HIGHEST,
                   preferred_element_type=jnp.float32))
    Nrm = jnp.sqrt(X * X + Y * Y)
    R3 = (P[:, None, :]
          + X[..., None] * w0p[None, None, :]
          + Y[..., None] * w1p[None, None, :]
          + Nrm[..., None] * wns[None, None, :]
          + bb[None, None, :])
    R3 = jnp.maximum(R3, 0.0)
    out_ref[0] = jnp.concatenate([NF, R3], axis=-1)

    n = jnp.float32(K9)
    sx = jnp.sum(X, axis=1)
    sy = jnp.sum(Y, axis=1)
    sxy = jnp.sum(X * Y, axis=1)
    sxx = jnp.sum(X * X, axis=1)
    m = (n * sxy - sx * sy) / (n * sxx - sx * sx + 1e-8)
    xm = X - (sx / n)[:, None]
    ym = Y - (sy / n)[:, None]
    r = jnp.sum(xm * ym, axis=1) / (
        jnp.sqrt(jnp.sum(xm * xm, axis=1) * jnp.sum(ym * ym, axis=1)) + 1e-8)
    ggf_ref[0, 0, :] = m
    ggf_ref[0, 1, :] = 1.0 - r


def _mlp_tc(g, Wx, Wy, wv):
    grid = (B, N // QC)
    g_spec = pl.BlockSpec((1, QC, K9, _TW), lambda b, q: (b, q, 0, 0))
    wxy_spec = pl.BlockSpec((K9, F), lambda b, q: (0, 0))
    wv_spec = pl.BlockSpec((4, F), lambda b, q: (0, 0))
    out_spec = pl.BlockSpec((1, QC, K9, 2 * F), lambda b, q: (b, q, 0, 0))
    ggf_spec = pl.BlockSpec((1, 2, QC), lambda b, q: (b, 0, q))
    return pl.pallas_call(
        _mlp_body,
        grid=grid,
        in_specs=[g_spec, wxy_spec, wxy_spec, wv_spec],
        out_specs=[out_spec, ggf_spec],
        out_shape=[
            jax.ShapeDtypeStruct((B, N, K9, 2 * F), jnp.float32),
            jax.ShapeDtypeStruct((B, 2, N), jnp.float32),
        ],
    )(g, Wx, Wy, wv)


def kernel(pc, feats, W, b):
    x = pc[..., 0]
    y = pc[..., 1]

    idx9 = _knn_tc(x, y, pc)

    offs = (jnp.arange(B, dtype=jnp.int32) * N)[:, None, None]
    gidx = (idx9 + offs).reshape(-1)
    table = jnp.concatenate(
        [feats, pc, jnp.zeros((B, N, _TW - F - 2), jnp.float32)], axis=-1)
    g = _gather_sc(table.reshape(B * N, _TW), gidx)
    g = g.reshape(B, N, K9, _TW)

    Wx = W[2::3]
    Wy = W[3::3]
    wv = jnp.stack([
        W[0] - jnp.sum(Wx, axis=0),
        W[1] - jnp.sum(Wy, axis=0),
        jnp.sum(W[4::3], axis=0),
        b,
    ])

    out, ggf2 = _mlp_tc(g, Wx, Wy, wv)
    ggf = jnp.moveaxis(ggf2, 1, 2)[:, :, None, :]
    return out, ggf


# speedup vs baseline: 16.3997x; 1.1008x over previous
import functools

import jax
import jax.numpy as jnp
from jax import lax
from jax.experimental import pallas as pl
from jax.experimental.pallas import tpu as pltpu
from jax.experimental.pallas import tpu_sc as plsc

K9 = 9
N = 4096
B = 4
F = 32
QB = 256
QC = 512


def _knn_body(xq_ref, yq_ref, xk_ref, yk_ref, qp_ref, kp_ref, idx_ref):
    qx = xq_ref[0, 0, :]
    qy = yq_ref[0, 0, :]
    kx = xk_ref[0, 0, :]
    ky = yk_ref[0, 0, :]

    qsq = qx * qx + qy * qy
    ksq = kx * kx + ky * ky
    dot = lax.dot_general(qp_ref[0], kp_ref[0],
                          (((1,), (1,)), ((), ())),
                          preferred_element_type=jnp.float32)
    d2 = qsq[:, None] + ksq[None, :] - 2.0 * dot

    fiota = lax.broadcasted_iota(jnp.int32, (QB, N), 1).astype(jnp.float32)
    inf = jnp.float32(jnp.inf)
    fn = jnp.float32(N)
    cur = d2
    for r in range(K9):
        m = jnp.min(cur, axis=1, keepdims=True)
        j = jnp.min(jnp.where(cur == m, fiota, fn), axis=1, keepdims=True)
        idx_ref[0, :, r] = j[:, 0].astype(jnp.int32)
        if r + 1 < K9:
            cur = jnp.where(fiota == j, inf, cur)


def _knn_tc(x, y, pc):
    x = x.reshape(B, 1, N)
    y = y.reshape(B, 1, N)
    grid = (B, N // QB)
    q_spec = pl.BlockSpec((1, 1, QB), lambda b, q: (b, 0, q))
    k_spec = pl.BlockSpec((1, 1, N), lambda b, q: (b, 0, 0))
    qp_spec = pl.BlockSpec((1, QB, 2), lambda b, q: (b, q, 0))
    kp_spec = pl.BlockSpec((1, N, 2), lambda b, q: (b, 0, 0))
    o_spec = pl.BlockSpec((1, QB, K9), lambda b, q: (b, q, 0))
    return pl.pallas_call(
        _knn_body,
        grid=grid,
        in_specs=[q_spec, q_spec, k_spec, k_spec, qp_spec, kp_spec],
        out_specs=o_spec,
        out_shape=jax.ShapeDtypeStruct((B, N, K9), jnp.int32),
    )(x, y, x, y, pc, pc)


_NC = 2
_NS = 16
_NW = _NC * _NS
_ROWS = B * N * K9
_RPW = _ROWS // _NW
_CH = 1152
_TW = 48


@functools.cache
def _make_gather_sc():
    @functools.partial(
        pl.kernel,
        out_type=jax.ShapeDtypeStruct((_ROWS, _TW), jnp.float32),
        mesh=plsc.VectorSubcoreMesh(core_axis_name="c", subcore_axis_name="s",
                                    num_cores=_NC, num_subcores=_NS),
        scratch_types=[
            pltpu.VMEM((_CH,), jnp.int32),
            pltpu.VMEM((_CH, _TW), jnp.float32),
            pltpu.SemaphoreType.DMA,
        ],
        compiler_params=pltpu.CompilerParams(use_tc_tiling_on_sc=False),
    )
    def gather(tab_hbm, idx_hbm, out_hbm, idx_v, rows_v, sem):
        wid = lax.axis_index("s") * _NC + lax.axis_index("c")
        base = wid * _RPW
        for c in range(_RPW // _CH):
            off = base + c * _CH
            pltpu.sync_copy(idx_hbm.at[pl.ds(off, _CH)], idx_v)
            pltpu.async_copy(tab_hbm.at[idx_v], rows_v, sem).wait()
            pltpu.sync_copy(rows_v, out_hbm.at[pl.ds(off, _CH)])

    return gather


def _gather_sc(tab, idx):
    return _make_gather_sc()(tab, idx)


def _mlp_body(g_ref, wx_ref, wy_ref, wv_ref, out_ref, ggf_ref):
    G = g_ref[0]
    NF = G[:, :, :F]
    X = G[:, :, F]
    Y = G[:, :, F + 1]
    Wx = wx_ref[...]
    Wy = wy_ref[...]
    w0p = wv_ref[0]
    w1p = wv_ref[1]
    wns = wv_ref[2]
    bb = wv_ref[3]

    P = (lax.dot(X, Wx, precision=lax.Precision.HIGHEST,
                 preferred_element_type=jnp.float32)
         + lax.dot(Y, Wy, precision=lax.Precision.HIGHEST,
                   preferred_element_type=jnp.float32))
    Nrm = jnp.sqrt(X * X + Y * Y)
    R3 = (P[:, None, :]
          + X[..., None] * w0p[None, None, :]
          + Y[..., None] * w1p[None, None, :]
          + Nrm[..., None] * wns[None, None, :]
          + bb[None, None, :])
    R3 = jnp.maximum(R3, 0.0)
    out_ref[0] = jnp.concatenate([NF, R3], axis=-1)

    n = jnp.float32(K9)
    sx = jnp.sum(X, axis=1)
    sy = jnp.sum(Y, axis=1)
    sxy = jnp.sum(X * Y, axis=1)
    sxx = jnp.sum(X * X, axis=1)
    m = (n * sxy - sx * sy) / (n * sxx - sx * sx + 1e-8)
    xm = X - (sx / n)[:, None]
    ym = Y - (sy / n)[:, None]
    r = jnp.sum(xm * ym, axis=1) / (
        jnp.sqrt(jnp.sum(xm * xm, axis=1) * jnp.sum(ym * ym, axis=1)) + 1e-8)
    ggf_ref[0, 0, :] = m
    ggf_ref[0, 1, :] = 1.0 - r


def _mlp_tc(g, Wx, Wy, wv):
    grid = (B, N // QC)
    g_spec = pl.BlockSpec((1, QC, K9, _TW), lambda b, q: (b, q, 0, 0))
    wxy_spec = pl.BlockSpec((K9, F), lambda b, q: (0, 0))
    wv_spec = pl.BlockSpec((4, F), lambda b, q: (0, 0))
    out_spec = pl.BlockSpec((1, QC, K9, 2 * F), lambda b, q: (b, q, 0, 0))
    ggf_spec = pl.BlockSpec((1, 2, QC), lambda b, q: (b, 0, q))
    return pl.pallas_call(
        _mlp_body,
        grid=grid,
        in_specs=[g_spec, wxy_spec, wxy_spec, wv_spec],
        out_specs=[out_spec, ggf_spec],
        out_shape=[
            jax.ShapeDtypeStruct((B, N, K9, 2 * F), jnp.float32),
            jax.ShapeDtypeStruct((B, 2, N), jnp.float32),
        ],
    )(g, Wx, Wy, wv)


def kernel(pc, feats, W, b):
    x = pc[..., 0]
    y = pc[..., 1]

    idx9 = _knn_tc(x, y, pc)

    offs = (jnp.arange(B, dtype=jnp.int32) * N)[:, None, None]
    gidx = (idx9 + offs).reshape(-1)
    table = jnp.concatenate(
        [feats, pc, jnp.zeros((B, N, _TW - F - 2), jnp.float32)], axis=-1)
    g = _gather_sc(table.reshape(B * N, _TW), gidx)
    g = g.reshape(B, N, K9, _TW)

    Wx = W[2::3]
    Wy = W[3::3]
    wv = jnp.stack([
        W[0] - jnp.sum(Wx, axis=0),
        W[1] - jnp.sum(Wy, axis=0),
        jnp.sum(W[4::3], axis=0),
        b,
    ])

    out, ggf2 = _mlp_tc(g, Wx, Wy, wv)
    ggf = jnp.moveaxis(ggf2, 1, 2)[:, :, None, :]
    return out, ggf
